# Initial kernel scaffold; baseline (speedup 1.0000x reference)
import jax
import jax.numpy as jnp
from jax.experimental import pallas as pl


def kernel(x, edge_index, W1, b1, W2, b2):
    raise NotImplementedError("write your pallas kernel here")


# speedup vs baseline: 22.3811x; 22.3811x over previous
import functools

import jax
import jax.numpy as jnp
from jax import lax
from jax.experimental import pallas as pl
from jax.experimental.pallas import tpu as pltpu
from jax.experimental.pallas import tpu_sc as plsc

N = 10000
D_IN = 128
D_OUT = 200
E = 320000

NC = 2
NS = 16
NW = NC * NS
CH = 128
CPT = 79
E_TILE = CPT * CH
E_PAD = NW * E_TILE
N_PAD = 10240
R_TILE = N_PAD // NS
D_PAD = 256

_mesh = plsc.VectorSubcoreMesh(core_axis_name="c", subcore_axis_name="s")


@functools.partial(
    pl.kernel, mesh=_mesh,
    out_type=jax.ShapeDtypeStruct((NC, N_PAD), jnp.float32),
    scratch_types=[
        pltpu.VMEM((CPT, CH), jnp.int32),
        pltpu.VMEM((CH,), jnp.float32),
        pltpu.VMEM_SHARED((N_PAD,), jnp.float32),
    ],
)
def _hist_k(dst3_hbm, ones_hbm, zeros1_hbm, deg_hbm, dst_v, ones_v, deg_sh):
    cid = lax.axis_index("c")
    sid = lax.axis_index("s")
    wid = sid * NC + cid
    r0 = sid * R_TILE
    pltpu.sync_copy(zeros1_hbm.at[pl.ds(r0, R_TILE)], deg_sh.at[pl.ds(r0, R_TILE)])
    pltpu.sync_copy(dst3_hbm.at[wid], dst_v)
    pltpu.sync_copy(ones_hbm, ones_v)
    plsc.subcore_barrier()

    def body(j, carry):
        pltpu.sync_copy(ones_v, deg_sh.at[dst_v.at[j]], add=True)
        return carry

    lax.fori_loop(0, CPT, body, 0)
    plsc.subcore_barrier()
    pltpu.sync_copy(deg_sh.at[pl.ds(r0, R_TILE)], deg_hbm.at[cid, pl.ds(r0, R_TILE)])


@functools.partial(
    pl.kernel, mesh=_mesh,
    out_type=jax.ShapeDtypeStruct((NC, N_PAD, D_IN), jnp.float32),
    scratch_types=[
        pltpu.VMEM((CPT, CH), jnp.int32),
        pltpu.VMEM((CPT, CH), jnp.int32),
        pltpu.VMEM((CH, D_IN), jnp.float32),
        pltpu.VMEM_SHARED((N_PAD, D_IN), jnp.float32),
        pltpu.SemaphoreType.DMA,
    ],
)
def _scatter_k(src3_hbm, dst3_hbm, g_hbm, zeros_hbm, out_hbm,
               src_v, dst_v, rows_v, acc_sh, sem):
    cid = lax.axis_index("c")
    sid = lax.axis_index("s")
    wid = sid * NC + cid
    r0 = sid * R_TILE

    @pl.when(cid == 0)
    def _():
        pltpu.sync_copy(g_hbm.at[pl.ds(r0, R_TILE)], acc_sh.at[pl.ds(r0, R_TILE)])

    @pl.when(cid != 0)
    def _():
        pltpu.sync_copy(zeros_hbm.at[pl.ds(r0, R_TILE)], acc_sh.at[pl.ds(r0, R_TILE)])

    pltpu.sync_copy(src3_hbm.at[wid], src_v)
    pltpu.sync_copy(dst3_hbm.at[wid], dst_v)
    plsc.subcore_barrier()

    def body(j, carry):
        pltpu.async_copy(g_hbm.at[src_v.at[j]], rows_v, sem).wait()
        pltpu.sync_copy(rows_v, acc_sh.at[dst_v.at[j]], add=True)
        return carry

    lax.fori_loop(0, CPT, body, 0)
    plsc.subcore_barrier()
    pltpu.sync_copy(acc_sh.at[pl.ds(r0, R_TILE)],
                    out_hbm.at[cid, pl.ds(r0, R_TILE)])


def _scale_body(p0_ref, p1_ref, x_ref, g_ref, dinv_ref):
    deg = p0_ref[...] + p1_ref[...] + 1.0
    dinv = lax.rsqrt(deg)
    dinv_ref[...] = dinv
    g_ref[...] = dinv * x_ref[...]


def _mm_body(q0_ref, q1_ref, dinv_ref, w1_ref, b1_ref, w2_ref, b2_ref,
             mu_ref, ls_ref):
    a = (q0_ref[...] + q1_ref[...]) * dinv_ref[...]
    mu_ref[...] = jnp.dot(a, w1_ref[...], preferred_element_type=jnp.float32) + b1_ref[...]
    ls_ref[...] = jnp.dot(a, w2_ref[...], preferred_element_type=jnp.float32) + b2_ref[...]


def kernel(x, edge_index, W1, b1, W2, b2):
    src = edge_index[0]
    dst = edge_index[1]
    pad_e = E_PAD - E
    src_p = jnp.concatenate([src, jnp.zeros((pad_e,), jnp.int32)])
    dst_p = jnp.concatenate([dst, jnp.full((pad_e,), N_PAD - 8, jnp.int32)])
    src3 = src_p.reshape(NW, CPT, CH)
    dst3 = dst_p.reshape(NW, CPT, CH)

    ones = jnp.ones((CH,), jnp.float32)
    zeros1 = jnp.zeros((N_PAD,), jnp.float32)
    zeros2 = jnp.zeros((N_PAD, D_IN), jnp.float32)
    x_pad = jnp.concatenate([x, jnp.zeros((N_PAD - N, D_IN), jnp.float32)])

    deg_p = _hist_k(dst3, ones, zeros1)
    p0c = deg_p[0].reshape(N_PAD, 1)
    p1c = deg_p[1].reshape(N_PAD, 1)

    BR = 1024
    g, dinv = pl.pallas_call(
        _scale_body,
        grid=(N_PAD // BR,),
        in_specs=[
            pl.BlockSpec((BR, 1), lambda i: (i, 0)),
            pl.BlockSpec((BR, 1), lambda i: (i, 0)),
            pl.BlockSpec((BR, D_IN), lambda i: (i, 0)),
        ],
        out_specs=[
            pl.BlockSpec((BR, D_IN), lambda i: (i, 0)),
            pl.BlockSpec((BR, 1), lambda i: (i, 0)),
        ],
        out_shape=[
            jax.ShapeDtypeStruct((N_PAD, D_IN), jnp.float32),
            jax.ShapeDtypeStruct((N_PAD, 1), jnp.float32),
        ],
    )(p0c, p1c, x_pad)

    parts = _scatter_k(src3, dst3, g, zeros2)

    w1p = jnp.zeros((D_IN, D_PAD), jnp.float32).at[:, :D_OUT].set(W1)
    w2p = jnp.zeros((D_IN, D_PAD), jnp.float32).at[:, :D_OUT].set(W2)
    b1p = jnp.zeros((1, D_PAD), jnp.float32).at[0, :D_OUT].set(b1)
    b2p = jnp.zeros((1, D_PAD), jnp.float32).at[0, :D_OUT].set(b2)

    mu_p, ls_p = pl.pallas_call(
        _mm_body,
        grid=(N_PAD // BR,),
        in_specs=[
            pl.BlockSpec((BR, D_IN), lambda i: (i, 0)),
            pl.BlockSpec((BR, D_IN), lambda i: (i, 0)),
            pl.BlockSpec((BR, 1), lambda i: (i, 0)),
            pl.BlockSpec((D_IN, D_PAD), lambda i: (0, 0)),
            pl.BlockSpec((1, D_PAD), lambda i: (0, 0)),
            pl.BlockSpec((D_IN, D_PAD), lambda i: (0, 0)),
            pl.BlockSpec((1, D_PAD), lambda i: (0, 0)),
        ],
        out_specs=[
            pl.BlockSpec((BR, D_PAD), lambda i: (i, 0)),
            pl.BlockSpec((BR, D_PAD), lambda i: (i, 0)),
        ],
        out_shape=[
            jax.ShapeDtypeStruct((N_PAD, D_PAD), jnp.float32),
            jax.ShapeDtypeStruct((N_PAD, D_PAD), jnp.float32),
        ],
    )(parts[0], parts[1], dinv, w1p, b1p, w2p, b2p)

    return (mu_p[:N, :D_OUT], ls_p[:N, :D_OUT])


# speedup vs baseline: 28.5585x; 1.2760x over previous
import functools

import jax
import jax.numpy as jnp
from jax import lax
from jax.experimental import pallas as pl
from jax.experimental.pallas import tpu as pltpu
from jax.experimental.pallas import tpu_sc as plsc

N = 10000
D_IN = 128
D_OUT = 200
E = 320000

NC = 2
NS = 16
NW = NC * NS
CHK = 80
CPT = 125
D_PAD = 256
R_STEP = 624
R_LEN = 640
N1 = 10240
R1D = N1 // NS

_mesh = plsc.VectorSubcoreMesh(core_axis_name="c", subcore_axis_name="s")


@functools.partial(
    pl.kernel, mesh=_mesh,
    out_type=jax.ShapeDtypeStruct((NC * N1,), jnp.float32),
    scratch_types=[
        pltpu.VMEM((CPT, CHK), jnp.int32),
        pltpu.VMEM((CHK,), jnp.float32),
        pltpu.VMEM_SHARED((N1,), jnp.float32),
    ],
)
def _hist_k(dst3_hbm, ones_hbm, zeros1_hbm, deg_hbm, dst_v, ones_v, deg_sh):
    cid = lax.axis_index("c")
    sid = lax.axis_index("s")
    wid = sid * NC + cid
    r0 = sid * R1D
    pltpu.sync_copy(zeros1_hbm.at[pl.ds(r0, R1D)], deg_sh.at[pl.ds(r0, R1D)])
    pltpu.sync_copy(dst3_hbm.at[wid], dst_v)
    pltpu.sync_copy(ones_hbm, ones_v)
    plsc.subcore_barrier()

    def body(j, carry):
        pltpu.sync_copy(ones_v, deg_sh.at[dst_v.at[j]], add=True)
        return carry

    lax.fori_loop(0, CPT, body, 0)
    plsc.subcore_barrier()
    pltpu.sync_copy(deg_sh.at[pl.ds(r0, R1D)],
                    deg_hbm.at[pl.ds(cid * N1 + r0, R1D)])


@functools.partial(
    pl.kernel, mesh=_mesh,
    out_type=jax.ShapeDtypeStruct((NC * N, D_IN), jnp.float32),
    scratch_types=[
        pltpu.VMEM((CPT, CHK), jnp.int32),
        pltpu.VMEM((CPT, CHK), jnp.int32),
        pltpu.VMEM((CHK, D_IN), jnp.float32),
        pltpu.VMEM_SHARED((N, D_IN), jnp.float32),
        pltpu.SemaphoreType.DMA,
    ],
)
def _scatter_k(src3_hbm, dst3_hbm, g_hbm, zeros_hbm, out_hbm,
               src_v, dst_v, rows_a, acc_sh, sem_a):
    cid = lax.axis_index("c")
    sid = lax.axis_index("s")
    wid = sid * NC + cid
    r0 = sid * R_STEP

    @pl.when(cid == 0)
    def _():
        pltpu.sync_copy(g_hbm.at[pl.ds(r0, R_LEN)], acc_sh.at[pl.ds(r0, R_LEN)])

    @pl.when(cid != 0)
    def _():
        pltpu.sync_copy(zeros_hbm.at[pl.ds(r0, R_LEN)], acc_sh.at[pl.ds(r0, R_LEN)])

    pltpu.sync_copy(src3_hbm.at[wid], src_v)
    pltpu.sync_copy(dst3_hbm.at[wid], dst_v)
    plsc.subcore_barrier()

    def fire(j, rows, sem):
        pltpu.async_copy(g_hbm.at[src_v.at[j]], rows, sem)

    def drain(rows, sem):
        pltpu.make_async_copy(g_hbm.at[pl.ds(0, CHK)], rows, sem).wait()

    def scat(j, rows):
        pltpu.sync_copy(rows, acc_sh.at[dst_v.at[j]], add=True)

    def body(j, carry):
        fire(j, rows_a, sem_a)
        drain(rows_a, sem_a)
        scat(j, rows_a)
        return carry

    lax.fori_loop(0, CPT, body, 0)

    plsc.subcore_barrier()
    pltpu.sync_copy(acc_sh.at[pl.ds(r0, R_LEN)],
                    out_hbm.at[pl.ds(cid * N + r0, R_LEN)])


def _scale_body(p0_ref, p1_ref, x_ref, g_ref, dinv_ref):
    deg = p0_ref[...] + p1_ref[...] + 1.0
    dinv = lax.rsqrt(deg)
    dinv_ref[...] = dinv
    g_ref[...] = dinv * x_ref[...]


def _mm_body(q0_ref, q1_ref, dinv_ref, w1_ref, b1_ref, w2_ref, b2_ref,
             mu_ref, ls_ref):
    a = (q0_ref[...] + q1_ref[...]) * dinv_ref[...]
    mu_ref[...] = jnp.dot(a, w1_ref[...], preferred_element_type=jnp.float32) + b1_ref[...]
    ls_ref[...] = jnp.dot(a, w2_ref[...], preferred_element_type=jnp.float32) + b2_ref[...]


def kernel(x, edge_index, W1, b1, W2, b2):
    src3 = edge_index[0].reshape(NW, CPT, CHK)
    dst3 = edge_index[1].reshape(NW, CPT, CHK)

    ones = jnp.ones((CHK,), jnp.float32)
    zeros1 = jnp.zeros((N1,), jnp.float32)
    zeros2 = jnp.zeros((N, D_IN), jnp.float32)

    deg_p = _hist_k(dst3, ones, zeros1)
    p0c = deg_p[:N].reshape(N, 1)
    p1c = deg_p[N1:N1 + N].reshape(N, 1)

    BR = 1000
    g, dinv = pl.pallas_call(
        _scale_body,
        grid=(N // BR,),
        in_specs=[
            pl.BlockSpec((BR, 1), lambda i: (i, 0)),
            pl.BlockSpec((BR, 1), lambda i: (i, 0)),
            pl.BlockSpec((BR, D_IN), lambda i: (i, 0)),
        ],
        out_specs=[
            pl.BlockSpec((BR, D_IN), lambda i: (i, 0)),
            pl.BlockSpec((BR, 1), lambda i: (i, 0)),
        ],
        out_shape=[
            jax.ShapeDtypeStruct((N, D_IN), jnp.float32),
            jax.ShapeDtypeStruct((N, 1), jnp.float32),
        ],
    )(p0c, p1c, x)

    parts = _scatter_k(src3, dst3, g, zeros2)

    w1p = jnp.zeros((D_IN, D_PAD), jnp.float32).at[:, :D_OUT].set(W1)
    w2p = jnp.zeros((D_IN, D_PAD), jnp.float32).at[:, :D_OUT].set(W2)
    b1p = jnp.zeros((1, D_PAD), jnp.float32).at[0, :D_OUT].set(b1)
    b2p = jnp.zeros((1, D_PAD), jnp.float32).at[0, :D_OUT].set(b2)

    mu_p, ls_p = pl.pallas_call(
        _mm_body,
        grid=(N // BR,),
        in_specs=[
            pl.BlockSpec((BR, D_IN), lambda i: (i, 0)),
            pl.BlockSpec((BR, D_IN), lambda i: (i, 0)),
            pl.BlockSpec((BR, 1), lambda i: (i, 0)),
            pl.BlockSpec((D_IN, D_PAD), lambda i: (0, 0)),
            pl.BlockSpec((1, D_PAD), lambda i: (0, 0)),
            pl.BlockSpec((D_IN, D_PAD), lambda i: (0, 0)),
            pl.BlockSpec((1, D_PAD), lambda i: (0, 0)),
        ],
        out_specs=[
            pl.BlockSpec((BR, D_PAD), lambda i: (i, 0)),
            pl.BlockSpec((BR, D_PAD), lambda i: (i, 0)),
        ],
        out_shape=[
            jax.ShapeDtypeStruct((N, D_PAD), jnp.float32),
            jax.ShapeDtypeStruct((N, D_PAD), jnp.float32),
        ],
    )(parts[:N], parts[N:], dinv, w1p, b1p, w2p, b2p)

    return (mu_p[:, :D_OUT], ls_p[:, :D_OUT])


# speedup vs baseline: 47.9348x; 1.6785x over previous
import functools

import jax
import jax.numpy as jnp
from jax import lax
from jax.experimental import pallas as pl
from jax.experimental.pallas import tpu as pltpu
from jax.experimental.pallas import tpu_sc as plsc

N = 10000
D_IN = 128
D_OUT = 200
E = 320000

NC = 2
NS = 16
NW = NC * NS
CHK = 80
CPT = 125
D_PAD = 256
R_STEP = 624
R_LEN = 640
N1 = 10240
R1D = N1 // NS
HBLK = 64

_mesh = plsc.VectorSubcoreMesh(core_axis_name="c", subcore_axis_name="s")


@functools.partial(
    pl.kernel, mesh=_mesh,
    out_type=jax.ShapeDtypeStruct((NC * N1,), jnp.float32),
    scratch_types=[
        pltpu.VMEM((CPT, CHK), jnp.int32),
        pltpu.VMEM((CHK,), jnp.float32),
        pltpu.VMEM_SHARED((N1,), jnp.float32),
    ],
)
def _hist_k(dst3_hbm, ones_hbm, zeros1_hbm, deg_hbm, dst_v, ones_v, deg_sh):
    cid = lax.axis_index("c")
    sid = lax.axis_index("s")
    wid = sid * NC + cid
    r0 = sid * R1D
    pltpu.sync_copy(zeros1_hbm.at[pl.ds(r0, R1D)], deg_sh.at[pl.ds(r0, R1D)])
    pltpu.sync_copy(dst3_hbm.at[wid], dst_v)
    pltpu.sync_copy(ones_hbm, ones_v)
    plsc.subcore_barrier()

    def body(j, carry):
        pltpu.sync_copy(ones_v, deg_sh.at[dst_v.at[j]], add=True)
        return carry

    lax.fori_loop(0, CPT, body, 0)
    plsc.subcore_barrier()
    pltpu.sync_copy(deg_sh.at[pl.ds(r0, R1D)],
                    deg_hbm.at[pl.ds(cid * N1 + r0, R1D)])


@functools.partial(
    pl.kernel, mesh=_mesh,
    out_type=jax.ShapeDtypeStruct((NC * N, D_IN), jnp.float32),
    scratch_types=[
        pltpu.VMEM((HBLK, CHK), jnp.int32),
        pltpu.VMEM((HBLK, CHK), jnp.int32),
        pltpu.VMEM((CHK, D_IN), jnp.float32),
        pltpu.VMEM((CHK, D_IN), jnp.float32),
        pltpu.VMEM_SHARED((N, D_IN), jnp.float32),
        pltpu.SemaphoreType.DMA,
        pltpu.SemaphoreType.DMA,
    ],
)
def _scatter_k(src3_hbm, dst3_hbm, g_hbm, zeros_hbm, out_hbm,
               src_v, dst_v, rows_a, rows_b, acc_sh, sem_a, sem_b):
    cid = lax.axis_index("c")
    sid = lax.axis_index("s")
    wid = sid * NC + cid
    r0 = sid * R_STEP

    @pl.when(cid == 0)
    def _():
        pltpu.sync_copy(g_hbm.at[pl.ds(r0, R_LEN)], acc_sh.at[pl.ds(r0, R_LEN)])

    @pl.when(cid != 0)
    def _():
        pltpu.sync_copy(zeros_hbm.at[pl.ds(r0, R_LEN)], acc_sh.at[pl.ds(r0, R_LEN)])

    plsc.subcore_barrier()

    def fire(j, rows, sem):
        pltpu.async_copy(g_hbm.at[src_v.at[j]], rows, sem)

    def drain(rows, sem):
        pltpu.make_async_copy(g_hbm.at[pl.ds(0, CHK)], rows, sem).wait()

    def scat(j, rows):
        pltpu.sync_copy(rows, acc_sh.at[dst_v.at[j]], add=True)

    def run_half(start, L):
        if L == HBLK:
            pltpu.sync_copy(src3_hbm.at[wid, pl.ds(start, HBLK)], src_v)
            pltpu.sync_copy(dst3_hbm.at[wid, pl.ds(start, HBLK)], dst_v)
        else:
            pltpu.sync_copy(src3_hbm.at[wid, pl.ds(start, L)],
                            src_v.at[pl.ds(0, L)])
            pltpu.sync_copy(dst3_hbm.at[wid, pl.ds(start, L)],
                            dst_v.at[pl.ds(0, L)])
        fire(0, rows_a, sem_a)
        fire(1, rows_b, sem_b)
        if L % 2 == 0:
            def body(k, carry):
                j = 2 * k
                drain(rows_a, sem_a)
                scat(j, rows_a)
                fire(j + 2, rows_a, sem_a)
                drain(rows_b, sem_b)
                scat(j + 1, rows_b)
                fire(j + 3, rows_b, sem_b)
                return carry
            lax.fori_loop(0, (L - 2) // 2, body, 0)
            drain(rows_a, sem_a)
            scat(L - 2, rows_a)
            drain(rows_b, sem_b)
            scat(L - 1, rows_b)
        else:
            def body(k, carry):
                j = 2 * k
                drain(rows_a, sem_a)
                scat(j, rows_a)
                fire(j + 2, rows_a, sem_a)
                drain(rows_b, sem_b)
                scat(j + 1, rows_b)
                fire(j + 3, rows_b, sem_b)
                return carry
            lax.fori_loop(0, (L - 3) // 2, body, 0)
            drain(rows_a, sem_a)
            scat(L - 3, rows_a)
            fire(L - 1, rows_a, sem_a)
            drain(rows_b, sem_b)
            scat(L - 2, rows_b)
            drain(rows_a, sem_a)
            scat(L - 1, rows_a)

    run_half(0, HBLK)
    run_half(HBLK, CPT - HBLK)

    plsc.subcore_barrier()
    pltpu.sync_copy(acc_sh.at[pl.ds(r0, R_LEN)],
                    out_hbm.at[pl.ds(cid * N + r0, R_LEN)])


def _scale_body(p0_ref, p1_ref, x_ref, g_ref, dinv_ref):
    deg = p0_ref[...] + p1_ref[...] + 1.0
    dinv = lax.rsqrt(deg)
    dinv_ref[...] = dinv
    g_ref[...] = dinv * x_ref[...]


def _mm_body(q0_ref, q1_ref, dinv_ref, w1_ref, b1_ref, w2_ref, b2_ref,
             mu_ref, ls_ref):
    a = (q0_ref[...] + q1_ref[...]) * dinv_ref[...]
    mu_ref[...] = jnp.dot(a, w1_ref[...], preferred_element_type=jnp.float32) + b1_ref[...]
    ls_ref[...] = jnp.dot(a, w2_ref[...], preferred_element_type=jnp.float32) + b2_ref[...]


_NB = N // 1000


def kernel(x, edge_index, W1, b1, W2, b2):
    src3 = edge_index[0].reshape(NW, CPT, CHK)
    dst3 = edge_index[1].reshape(NW, CPT, CHK)

    ones = jnp.ones((CHK,), jnp.float32)
    zeros1 = jnp.zeros((N1,), jnp.float32)
    zeros2 = jnp.zeros((N, D_IN), jnp.float32)

    deg_p = _hist_k(dst3, ones, zeros1)
    p0c = deg_p[:N].reshape(N, 1)
    p1c = deg_p[N1:N1 + N].reshape(N, 1)

    BR = 1000
    g, dinv = pl.pallas_call(
        _scale_body,
        grid=(N // BR,),
        in_specs=[
            pl.BlockSpec((BR, 1), lambda i: (i, 0)),
            pl.BlockSpec((BR, 1), lambda i: (i, 0)),
            pl.BlockSpec((BR, D_IN), lambda i: (i, 0)),
        ],
        out_specs=[
            pl.BlockSpec((BR, D_IN), lambda i: (i, 0)),
            pl.BlockSpec((BR, 1), lambda i: (i, 0)),
        ],
        out_shape=[
            jax.ShapeDtypeStruct((N, D_IN), jnp.float32),
            jax.ShapeDtypeStruct((N, 1), jnp.float32),
        ],
    )(p0c, p1c, x)

    parts = _scatter_k(src3, dst3, g, zeros2)

    nb = N // BR
    mu, ls = pl.pallas_call(
        _mm_body,
        grid=(nb,),
        in_specs=[
            pl.BlockSpec((BR, D_IN), lambda i: (i, 0)),
            pl.BlockSpec((BR, D_IN), lambda i: (i + N // 1000, 0)),
            pl.BlockSpec((BR, 1), lambda i: (i, 0)),
            pl.BlockSpec((D_IN, D_OUT), lambda i: (0, 0)),
            pl.BlockSpec((1, D_OUT), lambda i: (0, 0)),
            pl.BlockSpec((D_IN, D_OUT), lambda i: (0, 0)),
            pl.BlockSpec((1, D_OUT), lambda i: (0, 0)),
        ],
        out_specs=[
            pl.BlockSpec((BR, D_OUT), lambda i: (i, 0)),
            pl.BlockSpec((BR, D_OUT), lambda i: (i, 0)),
        ],
        out_shape=[
            jax.ShapeDtypeStruct((N, D_OUT), jnp.float32),
            jax.ShapeDtypeStruct((N, D_OUT), jnp.float32),
        ],
    )(parts, parts, dinv, W1, b1.reshape(1, D_OUT), W2, b2.reshape(1, D_OUT))

    return (mu, ls)


# speedup vs baseline: 50.2679x; 1.0487x over previous
import functools

import jax
import jax.numpy as jnp
from jax import lax
from jax.experimental import pallas as pl
from jax.experimental.pallas import tpu as pltpu
from jax.experimental.pallas import tpu_sc as plsc

N = 10000
D_IN = 128
D_OUT = 200
E = 320000

NC = 2
NS = 16
NW = NC * NS
CHK = 80
CPT = 125
D_PAD = 256
R_STEP = 624
R_LEN = 640
N1 = 10240
R1D = N1 // NS
HBLK = 64

_mesh = plsc.VectorSubcoreMesh(core_axis_name="c", subcore_axis_name="s")


@functools.partial(
    pl.kernel, mesh=_mesh,
    out_type=jax.ShapeDtypeStruct((NC * N1,), jnp.float32),
    scratch_types=[
        pltpu.VMEM((CPT, CHK), jnp.int32),
        pltpu.VMEM((CHK,), jnp.float32),
        pltpu.VMEM_SHARED((N1,), jnp.float32),
    ],
)
def _hist_k(ei3_hbm, ones_hbm, zeros1_hbm, deg_hbm, dst_v, ones_v, deg_sh):
    cid = lax.axis_index("c")
    sid = lax.axis_index("s")
    wid = sid * NC + cid
    r0 = sid * R1D
    pltpu.sync_copy(zeros1_hbm.at[pl.ds(r0, R1D)], deg_sh.at[pl.ds(r0, R1D)])
    pltpu.sync_copy(ei3_hbm.at[NW + wid], dst_v)
    pltpu.sync_copy(ones_hbm, ones_v)
    plsc.subcore_barrier()

    def body(j, carry):
        pltpu.sync_copy(ones_v, deg_sh.at[dst_v.at[j]], add=True)
        return carry

    lax.fori_loop(0, CPT, body, 0)
    plsc.subcore_barrier()
    pltpu.sync_copy(deg_sh.at[pl.ds(r0, R1D)],
                    deg_hbm.at[pl.ds(cid * N1 + r0, R1D)])


@functools.partial(
    pl.kernel, mesh=_mesh,
    out_type=jax.ShapeDtypeStruct((NC * N, D_IN), jnp.float32),
    scratch_types=[
        pltpu.VMEM((HBLK, CHK), jnp.int32),
        pltpu.VMEM((HBLK, CHK), jnp.int32),
        pltpu.VMEM((CHK, D_IN), jnp.float32),
        pltpu.VMEM((CHK, D_IN), jnp.float32),
        pltpu.VMEM_SHARED((N, D_IN), jnp.float32),
        pltpu.SemaphoreType.DMA,
        pltpu.SemaphoreType.DMA,
    ],
)
def _scatter_k(ei3_hbm, g_hbm, zeros_hbm, out_hbm,
               src_v, dst_v, rows_a, rows_b, acc_sh, sem_a, sem_b):
    cid = lax.axis_index("c")
    sid = lax.axis_index("s")
    wid = sid * NC + cid
    r0 = sid * R_STEP

    @pl.when(cid == 0)
    def _():
        pltpu.sync_copy(g_hbm.at[pl.ds(r0, R_LEN)], acc_sh.at[pl.ds(r0, R_LEN)])

    @pl.when(cid != 0)
    def _():
        pltpu.sync_copy(zeros_hbm.at[pl.ds(r0, R_LEN)], acc_sh.at[pl.ds(r0, R_LEN)])

    plsc.subcore_barrier()

    def fire(j, rows, sem):
        pltpu.async_copy(g_hbm.at[src_v.at[j]], rows, sem)

    def drain(rows, sem):
        pltpu.make_async_copy(g_hbm.at[pl.ds(0, CHK)], rows, sem).wait()

    def scat(j, rows):
        pltpu.sync_copy(rows, acc_sh.at[dst_v.at[j]], add=True)

    def run_half(start, L):
        if L == HBLK:
            pltpu.sync_copy(ei3_hbm.at[wid, pl.ds(start, HBLK)], src_v)
            pltpu.sync_copy(ei3_hbm.at[NW + wid, pl.ds(start, HBLK)], dst_v)
        else:
            pltpu.sync_copy(ei3_hbm.at[wid, pl.ds(start, L)],
                            src_v.at[pl.ds(0, L)])
            pltpu.sync_copy(ei3_hbm.at[NW + wid, pl.ds(start, L)],
                            dst_v.at[pl.ds(0, L)])
        fire(0, rows_a, sem_a)
        fire(1, rows_b, sem_b)
        if L % 2 == 0:
            def body(k, carry):
                j = 2 * k
                drain(rows_a, sem_a)
                scat(j, rows_a)
                fire(j + 2, rows_a, sem_a)
                drain(rows_b, sem_b)
                scat(j + 1, rows_b)
                fire(j + 3, rows_b, sem_b)
                return carry
            lax.fori_loop(0, (L - 2) // 2, body, 0)
            drain(rows_a, sem_a)
            scat(L - 2, rows_a)
            drain(rows_b, sem_b)
            scat(L - 1, rows_b)
        else:
            def body(k, carry):
                j = 2 * k
                drain(rows_a, sem_a)
                scat(j, rows_a)
                fire(j + 2, rows_a, sem_a)
                drain(rows_b, sem_b)
                scat(j + 1, rows_b)
                fire(j + 3, rows_b, sem_b)
                return carry
            lax.fori_loop(0, (L - 3) // 2, body, 0)
            drain(rows_a, sem_a)
            scat(L - 3, rows_a)
            fire(L - 1, rows_a, sem_a)
            drain(rows_b, sem_b)
            scat(L - 2, rows_b)
            drain(rows_a, sem_a)
            scat(L - 1, rows_a)

    run_half(0, HBLK)
    run_half(HBLK, CPT - HBLK)

    plsc.subcore_barrier()
    pltpu.sync_copy(acc_sh.at[pl.ds(r0, R_LEN)],
                    out_hbm.at[pl.ds(cid * N + r0, R_LEN)])


def _scale_body(p0_ref, p1_ref, x_ref, g_ref, dinv_ref):
    deg = p0_ref[...] + p1_ref[...] + 1.0
    dinv = lax.rsqrt(deg)
    dinv_ref[...] = dinv
    g_ref[...] = dinv * x_ref[...]


def _mm_body(q0_ref, q1_ref, dinv_ref, w1_ref, b1_ref, w2_ref, b2_ref,
             mu_ref, ls_ref):
    a = (q0_ref[...] + q1_ref[...]) * dinv_ref[...]
    mu_ref[...] = jnp.dot(a, w1_ref[...], preferred_element_type=jnp.float32) + b1_ref[...]
    ls_ref[...] = jnp.dot(a, w2_ref[...], preferred_element_type=jnp.float32) + b2_ref[...]


_NB = N // 1000


def kernel(x, edge_index, W1, b1, W2, b2):
    ei3 = edge_index.reshape(2 * NW, CPT, CHK)

    ones = jnp.ones((CHK,), jnp.float32)
    zeros1 = jnp.zeros((N1,), jnp.float32)
    zeros2 = jnp.zeros((N, D_IN), jnp.float32)

    deg_p = _hist_k(ei3, ones, zeros1)
    p0c = deg_p[:N].reshape(N, 1)
    p1c = deg_p[N1:N1 + N].reshape(N, 1)

    BR = 1000
    g, dinv = pl.pallas_call(
        _scale_body,
        grid=(N // BR,),
        in_specs=[
            pl.BlockSpec((BR, 1), lambda i: (i, 0)),
            pl.BlockSpec((BR, 1), lambda i: (i, 0)),
            pl.BlockSpec((BR, D_IN), lambda i: (i, 0)),
        ],
        out_specs=[
            pl.BlockSpec((BR, D_IN), lambda i: (i, 0)),
            pl.BlockSpec((BR, 1), lambda i: (i, 0)),
        ],
        out_shape=[
            jax.ShapeDtypeStruct((N, D_IN), jnp.float32),
            jax.ShapeDtypeStruct((N, 1), jnp.float32),
        ],
    )(p0c, p1c, x)

    parts = _scatter_k(ei3, g, zeros2)

    BRM = 2000
    mu, ls = pl.pallas_call(
        _mm_body,
        grid=(N // BRM,),
        in_specs=[
            pl.BlockSpec((BRM, D_IN), lambda i: (i, 0)),
            pl.BlockSpec((BRM, D_IN), lambda i: (i + N // 2000, 0)),
            pl.BlockSpec((BRM, 1), lambda i: (i, 0)),
            pl.BlockSpec((D_IN, D_OUT), lambda i: (0, 0)),
            pl.BlockSpec((1, D_OUT), lambda i: (0, 0)),
            pl.BlockSpec((D_IN, D_OUT), lambda i: (0, 0)),
            pl.BlockSpec((1, D_OUT), lambda i: (0, 0)),
        ],
        out_specs=[
            pl.BlockSpec((BRM, D_OUT), lambda i: (i, 0)),
            pl.BlockSpec((BRM, D_OUT), lambda i: (i, 0)),
        ],
        out_shape=[
            jax.ShapeDtypeStruct((N, D_OUT), jnp.float32),
            jax.ShapeDtypeStruct((N, D_OUT), jnp.float32),
        ],
    )(parts, parts, dinv, W1, b1.reshape(1, D_OUT), W2, b2.reshape(1, D_OUT))

    return (mu, ls)
